# baseline (speedup 1.0000x reference)
import functools

import jax
import jax.numpy as jnp
from jax import lax
from jax.experimental import pallas as pl
from jax.experimental.pallas import tpu as pltpu
from jax.experimental.pallas import tpu_sc as plsc

_N = 8
_C = 19
_H = 512
_W = 512
_K = 4096
_HB = 64
_NHB = _H // _HB

_P = _H * _W
_SPC = 4
_TPS = 8
_CHUNK = _P // _TPS
_NV = _CHUNK // 16
_NBKT = 256
_LVLS = (24, 16, 8, 0)


def _ce_kernel(pred_ref, tgt_ref, loss_ref):
    x = pred_ref[0]
    tgt = tgt_ref[0]
    m = x[0]
    for c in range(1, _C):
        m = jnp.maximum(m, x[c])
    s = jnp.zeros((_HB, _W), jnp.float32)
    xt = jnp.zeros((_HB, _W), jnp.float32)
    for c in range(_C):
        s = s + jnp.exp(x[c] - m)
        xt = xt + jnp.where(tgt == c, x[c], 0.0)
    nll = m + jnp.log(s) - xt
    loss_ref[0] = jnp.maximum(nll, 0.0)


def _make_ce(c0):
    return pl.pallas_call(
        _ce_kernel,
        grid=(_SPC, _NHB),
        in_specs=[
            pl.BlockSpec((1, _C, _HB, _W), lambda n, hb: (n + c0, 0, hb, 0)),
            pl.BlockSpec((1, _HB, _W), lambda n, hb: (n + c0, hb, 0)),
        ],
        out_specs=pl.BlockSpec((1, _HB, _W), lambda n, hb: (n, hb, 0)),
        out_shape=jax.ShapeDtypeStruct((_SPC, _H, _W), jnp.float32),
    )


def _sc_body(loss_hbm, out_hbm, data, hist, tmp, outv, fin, sh_hist, sh_fin):
    c = lax.axis_index("c")
    s = lax.axis_index("s")
    ln = s // _TPS
    t = s % _TPS
    n = c * 2 + ln

    lanes = lax.iota(jnp.int32, 16)
    zeros16 = jnp.zeros((16,), jnp.int32)
    ones16 = jnp.ones((16,), jnp.int32)

    pltpu.sync_copy(loss_hbm.at[n, pl.ds(t * _CHUNK, _CHUNK)], data)

    prefix = jnp.int32(0)
    k_tgt = jnp.int32(_K)
    c_gt = jnp.int32(0)

    for shift in _LVLS:
        @plsc.parallel_loop(0, _NBKT, unroll=8)
        def _zero(r):
            hist[pl.ds(r * 16, 16)] = zeros16

        first = shift == 24

        @plsc.parallel_loop(0, _NV, unroll=8)
        def _scan_data(i):
            x = data[pl.ds(i * 16, 16)]
            b = lax.bitcast_convert_type(x, jnp.int32)
            key = lax.shift_right_logical(b, shift) & 0xFF
            idx = (key << 4) | lanes
            if first:
                plsc.addupdate_scatter(hist, [idx], ones16)
            else:
                hi = lax.shift_right_logical(b, shift + 8)
                plsc.addupdate_scatter(hist, [idx], ones16,
                                       mask=hi == prefix)

        pltpu.sync_copy(hist, sh_hist.at[ln, t])
        plsc.subcore_barrier()
        for tt in range(_TPS):
            if tt == 0:
                pltpu.sync_copy(sh_hist.at[ln, 0], hist)
            else:
                pltpu.sync_copy(sh_hist.at[ln, tt], tmp)

                @plsc.parallel_loop(0, _NBKT, unroll=8)
                def _acc(r):
                    sl = pl.ds(r * 16, 16)
                    hist[sl] = hist[sl] + tmp[sl]

        def _scan_bkt(rr, carry):
            c_run, found, bkt, c_above = carry
            r = _NBKT - 1 - rr
            row_cnt = jnp.sum(hist[pl.ds(r * 16, 16)])
            newc = c_run + row_cnt
            crossing = jnp.logical_and(jnp.logical_not(found), newc >= k_tgt)
            bkt = jnp.where(crossing, r, bkt)
            c_above = jnp.where(crossing, c_run, c_above)
            c_run = jnp.where(found, c_run, newc)
            found = jnp.logical_or(found, crossing)
            return c_run, found, bkt, c_above

        _, _, bkt, c_above = lax.fori_loop(
            0, _NBKT, _scan_bkt,
            (jnp.int32(0), False, jnp.int32(0), jnp.int32(0)))

        prefix = (prefix << 8) | bkt
        k_tgt = k_tgt - c_above
        c_gt = c_gt + c_above
        plsc.subcore_barrier()

    t_val = lax.bitcast_convert_type(prefix, jnp.float32)

    zf = jnp.zeros((16,), jnp.float32)

    @plsc.parallel_loop(0, _NV, step=4, unroll=4, carry=(zf, zf, zf, zf))
    def _sum_gt(i, accs):
        out = []
        for j in range(4):
            x = data[pl.ds((i + j) * 16, 16)]
            out.append(accs[j] + jnp.where(x > t_val, x, 0.0))
        return tuple(out)

    a0, a1, a2, a3 = _sum_gt
    acc = (a0 + a1) + (a2 + a3)
    fin[pl.ds(0, 16)] = acc
    pltpu.sync_copy(fin, sh_fin.at[ln, t])
    plsc.subcore_barrier()

    @pl.when(t == 0)
    def _finish():
        tot = jnp.zeros((16,), jnp.float32)
        for tt in range(_TPS):
            pltpu.sync_copy(sh_fin.at[ln, tt], fin)
            tot = tot + fin[pl.ds(0, 16)]
        sum_gt = jnp.sum(tot)
        res = sum_gt + t_val * (_K - c_gt).astype(jnp.float32)
        outv[...] = jnp.full((16,), 0.0, jnp.float32) + res
        pltpu.sync_copy(outv, out_hbm.at[n])


@functools.partial(
    pl.kernel,
    out_type=jax.ShapeDtypeStruct((_SPC, 16), jnp.float32),
    mesh=plsc.VectorSubcoreMesh(core_axis_name="c", subcore_axis_name="s"),
    compiler_params=pltpu.CompilerParams(needs_layout_passes=False),
    scratch_types=[
        pltpu.VMEM((_CHUNK,), jnp.float32),
        pltpu.VMEM((_NBKT * 16,), jnp.int32),
        pltpu.VMEM((_NBKT * 16,), jnp.int32),
        pltpu.VMEM((16,), jnp.float32),
        pltpu.VMEM((4096,), jnp.float32),
        pltpu.VMEM_SHARED((2, _TPS, _NBKT * 16), jnp.int32),
        pltpu.VMEM_SHARED((2, _TPS, 4096), jnp.float32),
    ],
)
def _sc_select(loss_hbm, out_hbm, data, hist, tmp, outv, fin, sh_hist, sh_fin):
    _sc_body(loss_hbm, out_hbm, data, hist, tmp, outv, fin, sh_hist, sh_fin)


def kernel(predictions, targets):
    targets = targets.astype(jnp.int32)
    per = []
    for c0 in (0, _SPC):
        loss = _make_ce(c0)(predictions, targets)
        per.append(_sc_select(loss.reshape(_SPC, _P)))
    tot = jnp.sum(per[0][:, 0]) + jnp.sum(per[1][:, 0])
    return tot * (1.0 / (_K * _N))


# speedup vs baseline: 1.1538x; 1.1538x over previous
import functools

import jax
import jax.numpy as jnp
from jax import lax
from jax.experimental import pallas as pl
from jax.experimental.pallas import tpu as pltpu
from jax.experimental.pallas import tpu_sc as plsc

_N = 8
_C = 19
_H = 512
_W = 512
_K = 4096
_HB = 128
_NHB = _H // _HB

_P = _H * _W
_TPS = 4
_CHUNK = _P // _TPS
_NV = _CHUNK // 16
_NBKT = 256
_LVLS = (24, 16, 8, 0)


def _ce_kernel(pred_ref, tgt_ref, loss_ref):
    x = pred_ref[0]
    tgt = tgt_ref[0]
    m = x[0]
    for c in range(1, _C):
        m = jnp.maximum(m, x[c])
    s = jnp.zeros((_HB, _W), jnp.float32)
    xt = jnp.zeros((_HB, _W), jnp.float32)
    for c in range(_C):
        s = s + jnp.exp(x[c] - m)
        xt = xt + jnp.where(tgt == c, x[c], 0.0)
    nll = m + jnp.log(s) - xt
    loss_ref[0] = jnp.maximum(nll, 0.0)


def _ce(predictions, targets):
    return pl.pallas_call(
        _ce_kernel,
        grid=(_N, _NHB),
        in_specs=[
            pl.BlockSpec((1, _C, _HB, _W), lambda n, hb: (n, 0, hb, 0)),
            pl.BlockSpec((1, _HB, _W), lambda n, hb: (n, hb, 0)),
        ],
        out_specs=pl.BlockSpec((1, _HB, _W), lambda n, hb: (n, hb, 0)),
        out_shape=jax.ShapeDtypeStruct((_N, _H, _W), jnp.float32),
    )(predictions, targets)


def _sc_body(loss_hbm, out_hbm, data, hist, tmp, outv, fin, sh_hist, sh_fin):
    c = lax.axis_index("c")
    s = lax.axis_index("s")
    n = c * 4 + s // _TPS
    ln = s // _TPS
    t = s % _TPS

    lanes = lax.iota(jnp.int32, 16)
    zeros16 = jnp.zeros((16,), jnp.int32)
    ones16 = jnp.ones((16,), jnp.int32)

    pltpu.sync_copy(loss_hbm.at[n, pl.ds(t * _CHUNK, _CHUNK)], data)

    prefix = jnp.int32(0)
    k_tgt = jnp.int32(_K)
    c_gt = jnp.int32(0)

    for shift in _LVLS:
        @plsc.parallel_loop(0, _NBKT, unroll=8)
        def _zero(r):
            hist[pl.ds(r * 16, 16)] = zeros16

        first = shift == 24

        @plsc.parallel_loop(0, _NV, unroll=8)
        def _scan_data(i):
            x = data[pl.ds(i * 16, 16)]
            b = lax.bitcast_convert_type(x, jnp.int32)
            key = lax.shift_right_logical(b, shift) & 0xFF
            idx = (key << 4) | lanes
            if first:
                plsc.addupdate_scatter(hist, [idx], ones16)
            else:
                hi = lax.shift_right_logical(b, shift + 8)
                plsc.addupdate_scatter(hist, [idx], ones16,
                                       mask=hi == prefix)

        pltpu.sync_copy(hist, sh_hist.at[ln, t])
        plsc.subcore_barrier()
        for tt in range(_TPS):
            if tt == 0:
                pltpu.sync_copy(sh_hist.at[ln, 0], hist)
            else:
                pltpu.sync_copy(sh_hist.at[ln, tt], tmp)

                @plsc.parallel_loop(0, _NBKT, unroll=8)
                def _acc(r):
                    sl = pl.ds(r * 16, 16)
                    hist[sl] = hist[sl] + tmp[sl]

        def _scan_bkt(rr, carry):
            c_run, found, bkt, c_above = carry
            r = _NBKT - 1 - rr
            row_cnt = jnp.sum(hist[pl.ds(r * 16, 16)])
            newc = c_run + row_cnt
            crossing = jnp.logical_and(jnp.logical_not(found), newc >= k_tgt)
            bkt = jnp.where(crossing, r, bkt)
            c_above = jnp.where(crossing, c_run, c_above)
            c_run = jnp.where(found, c_run, newc)
            found = jnp.logical_or(found, crossing)
            return c_run, found, bkt, c_above

        _, _, bkt, c_above = lax.fori_loop(
            0, _NBKT, _scan_bkt,
            (jnp.int32(0), False, jnp.int32(0), jnp.int32(0)))

        prefix = (prefix << 8) | bkt
        k_tgt = k_tgt - c_above
        c_gt = c_gt + c_above
        plsc.subcore_barrier()

    t_val = lax.bitcast_convert_type(prefix, jnp.float32)

    zf = jnp.zeros((16,), jnp.float32)

    @plsc.parallel_loop(0, _NV, step=4, unroll=4, carry=(zf, zf, zf, zf))
    def _sum_gt(i, accs):
        out = []
        for j in range(4):
            x = data[pl.ds((i + j) * 16, 16)]
            out.append(accs[j] + jnp.where(x > t_val, x, 0.0))
        return tuple(out)

    a0, a1, a2, a3 = _sum_gt
    acc = (a0 + a1) + (a2 + a3)
    fin[pl.ds(0, 16)] = acc
    pltpu.sync_copy(fin, sh_fin.at[ln, t])
    plsc.subcore_barrier()

    @pl.when(t == 0)
    def _finish():
        tot = jnp.zeros((16,), jnp.float32)
        for tt in range(_TPS):
            pltpu.sync_copy(sh_fin.at[ln, tt], fin)
            tot = tot + fin[pl.ds(0, 16)]
        sum_gt = jnp.sum(tot)
        res = sum_gt + t_val * (_K - c_gt).astype(jnp.float32)
        outv[...] = jnp.full((16,), 0.0, jnp.float32) + res
        pltpu.sync_copy(outv, out_hbm.at[n])


@functools.partial(
    pl.kernel,
    out_type=jax.ShapeDtypeStruct((_N, 16), jnp.float32),
    mesh=plsc.VectorSubcoreMesh(core_axis_name="c", subcore_axis_name="s"),
    compiler_params=pltpu.CompilerParams(needs_layout_passes=False),
    scratch_types=[
        pltpu.VMEM((_CHUNK,), jnp.float32),
        pltpu.VMEM((_NBKT * 16,), jnp.int32),
        pltpu.VMEM((_NBKT * 16,), jnp.int32),
        pltpu.VMEM((16,), jnp.float32),
        pltpu.VMEM((4096,), jnp.float32),
        pltpu.VMEM_SHARED((4, _TPS, _NBKT * 16), jnp.int32),
        pltpu.VMEM_SHARED((4, _TPS, 4096), jnp.float32),
    ],
)
def _sc_select(loss_hbm, out_hbm, data, hist, tmp, outv, fin, sh_hist, sh_fin):
    _sc_body(loss_hbm, out_hbm, data, hist, tmp, outv, fin, sh_hist, sh_fin)


def kernel(predictions, targets):
    targets = targets.astype(jnp.int32)
    loss = _ce(predictions, targets)
    per = _sc_select(loss.reshape(_N, _P))
    return jnp.sum(per[:, 0]) * (1.0 / (_K * _N))


# speedup vs baseline: 1.2178x; 1.0555x over previous
import functools

import jax
import jax.numpy as jnp
from jax import lax
from jax.experimental import pallas as pl
from jax.experimental.pallas import tpu as pltpu
from jax.experimental.pallas import tpu_sc as plsc

_N = 8
_C = 19
_H = 512
_W = 512
_K = 4096
_HB = 256
_NHB = _H // _HB

_P = _H * _W
_TPS = 4
_CHUNK = _P // _TPS
_NV = _CHUNK // 16
_NBKT = 256
_LVLS = (24, 16, 8, 0)


def _ce_kernel(pred_ref, tgt_ref, loss_ref):
    x = pred_ref[0]
    tgt = tgt_ref[0]
    m = x[0]
    for c in range(1, _C):
        m = jnp.maximum(m, x[c])
    s = jnp.zeros((_HB, _W), jnp.float32)
    xt = jnp.zeros((_HB, _W), jnp.float32)
    for c in range(_C):
        s = s + jnp.exp(x[c] - m)
        xt = xt + jnp.where(tgt == c, x[c], 0.0)
    nll = m + jnp.log(s) - xt
    loss_ref[0] = jnp.maximum(nll, 0.0)


def _ce(predictions, targets):
    return pl.pallas_call(
        _ce_kernel,
        grid=(_N, _NHB),
        in_specs=[
            pl.BlockSpec((1, _C, _HB, _W), lambda n, hb: (n, 0, hb, 0)),
            pl.BlockSpec((1, _HB, _W), lambda n, hb: (n, hb, 0)),
        ],
        out_specs=pl.BlockSpec((1, _HB, _W), lambda n, hb: (n, hb, 0)),
        out_shape=jax.ShapeDtypeStruct((_N, _H, _W), jnp.float32),
    )(predictions, targets)


def _sc_body(loss_hbm, out_hbm, data, hist, tmp, outv, fin, sh_hist, sh_fin):
    c = lax.axis_index("c")
    s = lax.axis_index("s")
    n = c * 4 + s // _TPS
    ln = s // _TPS
    t = s % _TPS

    lanes = lax.iota(jnp.int32, 16)
    zeros16 = jnp.zeros((16,), jnp.int32)
    ones16 = jnp.ones((16,), jnp.int32)

    pltpu.sync_copy(loss_hbm.at[n, pl.ds(t * _CHUNK, _CHUNK)], data)

    prefix = jnp.int32(0)
    k_tgt = jnp.int32(_K)
    c_gt = jnp.int32(0)

    for shift in _LVLS:
        @plsc.parallel_loop(0, _NBKT, unroll=8)
        def _zero(r):
            hist[pl.ds(r * 16, 16)] = zeros16

        first = shift == 24

        @plsc.parallel_loop(0, _NV, unroll=8)
        def _scan_data(i):
            x = data[pl.ds(i * 16, 16)]
            b = lax.bitcast_convert_type(x, jnp.int32)
            key = lax.shift_right_logical(b, shift) & 0xFF
            idx = (key << 4) | lanes
            if first:
                plsc.addupdate_scatter(hist, [idx], ones16)
            else:
                hi = lax.shift_right_logical(b, shift + 8)
                plsc.addupdate_scatter(hist, [idx], ones16,
                                       mask=hi == prefix)

        pltpu.sync_copy(hist, sh_hist.at[ln, t])
        plsc.subcore_barrier()
        for tt in range(_TPS):
            if tt == 0:
                pltpu.sync_copy(sh_hist.at[ln, 0], hist)
            else:
                pltpu.sync_copy(sh_hist.at[ln, tt], tmp)

                @plsc.parallel_loop(0, _NBKT, unroll=8)
                def _acc(r):
                    sl = pl.ds(r * 16, 16)
                    hist[sl] = hist[sl] + tmp[sl]

        def _scan_bkt(rr, carry):
            c_run, found, bkt, c_above = carry
            r = _NBKT - 1 - rr
            row_cnt = jnp.sum(hist[pl.ds(r * 16, 16)])
            newc = c_run + row_cnt
            crossing = jnp.logical_and(jnp.logical_not(found), newc >= k_tgt)
            bkt = jnp.where(crossing, r, bkt)
            c_above = jnp.where(crossing, c_run, c_above)
            c_run = jnp.where(found, c_run, newc)
            found = jnp.logical_or(found, crossing)
            return c_run, found, bkt, c_above

        _, _, bkt, c_above = lax.fori_loop(
            0, _NBKT, _scan_bkt,
            (jnp.int32(0), False, jnp.int32(0), jnp.int32(0)))

        prefix = (prefix << 8) | bkt
        k_tgt = k_tgt - c_above
        c_gt = c_gt + c_above
        plsc.subcore_barrier()

    t_val = lax.bitcast_convert_type(prefix, jnp.float32)

    zf = jnp.zeros((16,), jnp.float32)

    @plsc.parallel_loop(0, _NV, step=4, unroll=4, carry=(zf, zf, zf, zf))
    def _sum_gt(i, accs):
        out = []
        for j in range(4):
            x = data[pl.ds((i + j) * 16, 16)]
            out.append(accs[j] + jnp.where(x > t_val, x, 0.0))
        return tuple(out)

    a0, a1, a2, a3 = _sum_gt
    acc = (a0 + a1) + (a2 + a3)
    fin[pl.ds(0, 16)] = acc
    pltpu.sync_copy(fin, sh_fin.at[ln, t])
    plsc.subcore_barrier()

    @pl.when(t == 0)
    def _finish():
        tot = jnp.zeros((16,), jnp.float32)
        for tt in range(_TPS):
            pltpu.sync_copy(sh_fin.at[ln, tt], fin)
            tot = tot + fin[pl.ds(0, 16)]
        sum_gt = jnp.sum(tot)
        res = sum_gt + t_val * (_K - c_gt).astype(jnp.float32)
        outv[...] = jnp.full((16,), 0.0, jnp.float32) + res
        pltpu.sync_copy(outv, out_hbm.at[n])


@functools.partial(
    pl.kernel,
    out_type=jax.ShapeDtypeStruct((_N, 16), jnp.float32),
    mesh=plsc.VectorSubcoreMesh(core_axis_name="c", subcore_axis_name="s"),
    compiler_params=pltpu.CompilerParams(needs_layout_passes=False),
    scratch_types=[
        pltpu.VMEM((_CHUNK,), jnp.float32),
        pltpu.VMEM((_NBKT * 16,), jnp.int32),
        pltpu.VMEM((_NBKT * 16,), jnp.int32),
        pltpu.VMEM((16,), jnp.float32),
        pltpu.VMEM((4096,), jnp.float32),
        pltpu.VMEM_SHARED((4, _TPS, _NBKT * 16), jnp.int32),
        pltpu.VMEM_SHARED((4, _TPS, 4096), jnp.float32),
    ],
)
def _sc_select(loss_hbm, out_hbm, data, hist, tmp, outv, fin, sh_hist, sh_fin):
    _sc_body(loss_hbm, out_hbm, data, hist, tmp, outv, fin, sh_hist, sh_fin)


def kernel(predictions, targets):
    targets = targets.astype(jnp.int32)
    loss = _ce(predictions, targets)
    per = _sc_select(loss.reshape(_N, _P))
    return jnp.sum(per[:, 0]) * (1.0 / (_K * _N))


# speedup vs baseline: 1.2286x; 1.0088x over previous
import functools

import jax
import jax.numpy as jnp
from jax import lax
from jax.experimental import pallas as pl
from jax.experimental.pallas import tpu as pltpu
from jax.experimental.pallas import tpu_sc as plsc

_N = 8
_C = 19
_H = 512
_W = 512
_K = 4096
_HB = 512
_NHB = _H // _HB

_P = _H * _W
_TPS = 4
_CHUNK = _P // _TPS
_NV = _CHUNK // 16
_NBKT = 256
_LVLS = (24, 16, 8, 0)


def _ce_kernel(pred_ref, tgt_ref, loss_ref):
    x = pred_ref[0]
    tgt = tgt_ref[0]
    m = x[0]
    for c in range(1, _C):
        m = jnp.maximum(m, x[c])
    s = jnp.zeros((_HB, _W), jnp.float32)
    xt = jnp.zeros((_HB, _W), jnp.float32)
    for c in range(_C):
        s = s + jnp.exp(x[c] - m)
        xt = xt + jnp.where(tgt == c, x[c], 0.0)
    nll = m + jnp.log(s) - xt
    loss_ref[0] = jnp.maximum(nll, 0.0)


def _ce(predictions, targets):
    return pl.pallas_call(
        _ce_kernel,
        grid=(_N, _NHB),
        in_specs=[
            pl.BlockSpec((1, _C, _HB, _W), lambda n, hb: (n, 0, hb, 0)),
            pl.BlockSpec((1, _HB, _W), lambda n, hb: (n, hb, 0)),
        ],
        out_specs=pl.BlockSpec((1, _HB, _W), lambda n, hb: (n, hb, 0)),
        out_shape=jax.ShapeDtypeStruct((_N, _H, _W), jnp.float32),
    )(predictions, targets)


def _sc_body(loss_hbm, out_hbm, data, hist, tmp, outv, fin, sh_hist, sh_fin):
    c = lax.axis_index("c")
    s = lax.axis_index("s")
    n = c * 4 + s // _TPS
    ln = s // _TPS
    t = s % _TPS

    lanes = lax.iota(jnp.int32, 16)
    zeros16 = jnp.zeros((16,), jnp.int32)
    ones16 = jnp.ones((16,), jnp.int32)

    pltpu.sync_copy(loss_hbm.at[n, pl.ds(t * _CHUNK, _CHUNK)], data)

    prefix = jnp.int32(0)
    k_tgt = jnp.int32(_K)
    c_gt = jnp.int32(0)

    for shift in _LVLS:
        @plsc.parallel_loop(0, _NBKT, unroll=8)
        def _zero(r):
            hist[pl.ds(r * 16, 16)] = zeros16

        first = shift == 24

        @plsc.parallel_loop(0, _NV, unroll=8)
        def _scan_data(i):
            x = data[pl.ds(i * 16, 16)]
            b = lax.bitcast_convert_type(x, jnp.int32)
            key = lax.shift_right_logical(b, shift) & 0xFF
            idx = (key << 4) | lanes
            if first:
                plsc.addupdate_scatter(hist, [idx], ones16)
            else:
                hi = lax.shift_right_logical(b, shift + 8)
                plsc.addupdate_scatter(hist, [idx], ones16,
                                       mask=hi == prefix)

        pltpu.sync_copy(hist, sh_hist.at[ln, t])
        plsc.subcore_barrier()
        for tt in range(_TPS):
            if tt == 0:
                pltpu.sync_copy(sh_hist.at[ln, 0], hist)
            else:
                pltpu.sync_copy(sh_hist.at[ln, tt], tmp)

                @plsc.parallel_loop(0, _NBKT, unroll=8)
                def _acc(r):
                    sl = pl.ds(r * 16, 16)
                    hist[sl] = hist[sl] + tmp[sl]

        def _scan_bkt(rr, carry):
            c_run, found, bkt, c_above = carry
            r = _NBKT - 1 - rr
            row_cnt = jnp.sum(hist[pl.ds(r * 16, 16)])
            newc = c_run + row_cnt
            crossing = jnp.logical_and(jnp.logical_not(found), newc >= k_tgt)
            bkt = jnp.where(crossing, r, bkt)
            c_above = jnp.where(crossing, c_run, c_above)
            c_run = jnp.where(found, c_run, newc)
            found = jnp.logical_or(found, crossing)
            return c_run, found, bkt, c_above

        _, _, bkt, c_above = lax.fori_loop(
            0, _NBKT, _scan_bkt,
            (jnp.int32(0), False, jnp.int32(0), jnp.int32(0)))

        prefix = (prefix << 8) | bkt
        k_tgt = k_tgt - c_above
        c_gt = c_gt + c_above
        plsc.subcore_barrier()

    t_val = lax.bitcast_convert_type(prefix, jnp.float32)

    zf = jnp.zeros((16,), jnp.float32)

    @plsc.parallel_loop(0, _NV, step=4, unroll=4, carry=(zf, zf, zf, zf))
    def _sum_gt(i, accs):
        out = []
        for j in range(4):
            x = data[pl.ds((i + j) * 16, 16)]
            out.append(accs[j] + jnp.where(x > t_val, x, 0.0))
        return tuple(out)

    a0, a1, a2, a3 = _sum_gt
    acc = (a0 + a1) + (a2 + a3)
    fin[pl.ds(0, 16)] = acc
    pltpu.sync_copy(fin, sh_fin.at[ln, t])
    plsc.subcore_barrier()

    @pl.when(t == 0)
    def _finish():
        tot = jnp.zeros((16,), jnp.float32)
        for tt in range(_TPS):
            pltpu.sync_copy(sh_fin.at[ln, tt], fin)
            tot = tot + fin[pl.ds(0, 16)]
        sum_gt = jnp.sum(tot)
        res = sum_gt + t_val * (_K - c_gt).astype(jnp.float32)
        outv[...] = jnp.full((16,), 0.0, jnp.float32) + res
        pltpu.sync_copy(outv, out_hbm.at[n])


@functools.partial(
    pl.kernel,
    out_type=jax.ShapeDtypeStruct((_N, 16), jnp.float32),
    mesh=plsc.VectorSubcoreMesh(core_axis_name="c", subcore_axis_name="s"),
    compiler_params=pltpu.CompilerParams(needs_layout_passes=False),
    scratch_types=[
        pltpu.VMEM((_CHUNK,), jnp.float32),
        pltpu.VMEM((_NBKT * 16,), jnp.int32),
        pltpu.VMEM((_NBKT * 16,), jnp.int32),
        pltpu.VMEM((16,), jnp.float32),
        pltpu.VMEM((4096,), jnp.float32),
        pltpu.VMEM_SHARED((4, _TPS, _NBKT * 16), jnp.int32),
        pltpu.VMEM_SHARED((4, _TPS, 4096), jnp.float32),
    ],
)
def _sc_select(loss_hbm, out_hbm, data, hist, tmp, outv, fin, sh_hist, sh_fin):
    _sc_body(loss_hbm, out_hbm, data, hist, tmp, outv, fin, sh_hist, sh_fin)


def kernel(predictions, targets):
    targets = targets.astype(jnp.int32)
    loss = _ce(predictions, targets)
    per = _sc_select(loss.reshape(_N, _P))
    return jnp.sum(per[:, 0]) * (1.0 / (_K * _N))
